# baseline (speedup 1.0000x reference)
import functools

import jax
import jax.numpy as jnp
from jax.experimental import pallas as pl
from jax.experimental.pallas import tpu as pltpu


def _fused_kernel(x_ref, L_ref, wc_ref, bc_ref, wfc_ref, bfc_ref, out_ref,
                  *, K, G):
    Lb2 = (L_ref[...] * 2.0).astype(jnp.bfloat16)
    dot = functools.partial(jnp.dot, preferred_element_type=jnp.float32)

    CHUNK = 12
    S = 2
    B = x_ref.shape[0]
    bs = B // S

    def wcb(k, g):
        return wc_ref[k, g].astype(jnp.bfloat16)

    def flush(pend, acc):
        for g in range(G):
            s = wcb(pend[0][0], g) * pend[0][1]
            for k, t in pend[1:]:
                s = s + wcb(k, g) * t
            acc[g] = s.astype(jnp.float32) if acc[g] is None else acc[g] + s

    t_m2, t_m1, tb_m1, acc, pend = [], [], [], [], []
    for s in range(S):
        x_s = x_ref[s * bs:(s + 1) * bs, :]
        xb = x_s.astype(jnp.bfloat16)
        t1 = 0.5 * dot(xb, Lb2)
        tb1 = t1.astype(jnp.bfloat16)
        t_m2.append(x_s)
        t_m1.append(t1)
        tb_m1.append(tb1)
        acc.append([None] * G)
        pend.append([(0, xb), (1, tb1)])

    for k in range(2, K):
        for s in range(S):
            t = dot(tb_m1[s], Lb2) - t_m2[s]
            t_m2[s], t_m1[s] = t_m1[s], t
            tb_m1[s] = t.astype(jnp.bfloat16)
            pend[s].append((k, tb_m1[s]))
            if len(pend[s]) == CHUNK:
                flush(pend[s], acc[s])
                pend[s] = []
    for s in range(S):
        if pend[s]:
            flush(pend[s], acc[s])

    wfcb = wfc_ref[...].astype(jnp.bfloat16)
    for s in range(S):
        logits = bfc_ref[...]
        for g in range(G):
            h_g = jnp.maximum(acc[s][g] + bc_ref[g], 0.0)
            logits = logits + dot(h_g.astype(jnp.bfloat16), wfcb[g])
        m = jnp.max(logits, axis=1, keepdims=True)
        sh = logits - m
        out_ref[s * bs:(s + 1) * bs, :] = sh - jnp.log(
            jnp.sum(jnp.exp(sh), axis=1, keepdims=True))


def kernel(x, L, W_cheb, b_cheb, W_fc, b_fc):
    B, N, F_IN = x.shape
    K, _, G = W_cheb.shape
    D = W_fc.shape[1]
    xt = x.reshape(B, N)
    wc = W_cheb.reshape(K, G)
    wfc = W_fc.reshape(N, G, D).transpose(1, 0, 2)

    fn = pl.pallas_call(
        functools.partial(_fused_kernel, K=K, G=G),
        out_shape=jax.ShapeDtypeStruct((B, D), jnp.float32),
        in_specs=[
            pl.BlockSpec(memory_space=pltpu.VMEM),
            pl.BlockSpec(memory_space=pltpu.VMEM),
            pl.BlockSpec(memory_space=pltpu.SMEM),
            pl.BlockSpec(memory_space=pltpu.SMEM),
            pl.BlockSpec(memory_space=pltpu.VMEM),
            pl.BlockSpec(memory_space=pltpu.VMEM),
        ],
        out_specs=pl.BlockSpec(memory_space=pltpu.VMEM),
    )
    return fn(xt, L, wc, b_cheb, wfc, b_fc.reshape(1, D))


# speedup vs baseline: 1.0095x; 1.0095x over previous
import functools

import jax
import jax.numpy as jnp
from jax.experimental import pallas as pl
from jax.experimental.pallas import tpu as pltpu


def _fused_kernel(x_ref, L_ref, wc_ref, bc_ref, wfc_ref, bfc_ref, out_ref,
                  *, K, G):
    Lb2 = (L_ref[...] * 2.0).astype(jnp.bfloat16)
    dot = functools.partial(jnp.dot, preferred_element_type=jnp.float32)

    CHUNK = 12
    t_m2 = x_ref[...]
    tb = x_ref[...].astype(jnp.bfloat16)
    t_m1 = 0.5 * dot(tb, Lb2)
    tb1 = t_m1.astype(jnp.bfloat16)
    acc = [None] * G
    pend = [(0, tb), (1, tb1)]

    def wcb(k, g):
        return wc_ref[k, g].astype(jnp.bfloat16)

    def flush(pend, acc):
        for g in range(G):
            s = wcb(pend[0][0], g) * pend[0][1]
            for k, t in pend[1:]:
                s = s + wcb(k, g) * t
            acc[g] = s.astype(jnp.float32) if acc[g] is None else acc[g] + s

    tb_m1 = tb1
    for k in range(2, K):
        t = dot(tb_m1, Lb2) - t_m2
        t_m2, t_m1 = t_m1, t
        tb_m1 = t.astype(jnp.bfloat16)
        pend.append((k, tb_m1))
        if len(pend) == CHUNK:
            flush(pend, acc)
            pend = []
    if pend:
        flush(pend, acc)

    wfcb = wfc_ref[...].astype(jnp.bfloat16)
    logits = bfc_ref[...]
    for g in range(G):
        h_g = jnp.maximum(acc[g] + bc_ref[g], 0.0)
        logits = logits + dot(h_g.astype(jnp.bfloat16), wfcb[g])

    m = jnp.max(logits, axis=1, keepdims=True)
    s = logits - m
    out_ref[...] = s - jnp.log(jnp.sum(jnp.exp(s), axis=1, keepdims=True))


def kernel(x, L, W_cheb, b_cheb, W_fc, b_fc):
    B, N, F_IN = x.shape
    K, _, G = W_cheb.shape
    D = W_fc.shape[1]
    xt = x.reshape(B, N)
    wc = W_cheb.reshape(K, G)
    wfc = W_fc.reshape(N, G, D).transpose(1, 0, 2)

    fn = pl.pallas_call(
        functools.partial(_fused_kernel, K=K, G=G),
        out_shape=jax.ShapeDtypeStruct((B, D), jnp.float32),
        in_specs=[
            pl.BlockSpec(memory_space=pltpu.VMEM),
            pl.BlockSpec(memory_space=pltpu.VMEM),
            pl.BlockSpec(memory_space=pltpu.SMEM),
            pl.BlockSpec(memory_space=pltpu.SMEM),
            pl.BlockSpec(memory_space=pltpu.VMEM),
            pl.BlockSpec(memory_space=pltpu.VMEM),
        ],
        out_specs=pl.BlockSpec(memory_space=pltpu.VMEM),
    )
    return fn(xt, L, wc, b_cheb, wfc, b_fc.reshape(1, D))
